# Initial kernel scaffold; baseline (speedup 1.0000x reference)
import jax
import jax.numpy as jnp
from jax.experimental import pallas as pl


def kernel(t, K):
    raise NotImplementedError("write your pallas kernel here")


# speedup vs baseline: 6.9854x; 6.9854x over previous
import jax
import jax.numpy as jnp
from jax import lax
from jax.experimental import pallas as pl
from jax.experimental.pallas import tpu as pltpu
from jax.experimental.pallas import tpu_sc as plsc

L = 16
C = 32768
NV = C // L
G = 16
ROWS = 1024
NW = 32
RPW = ROWS // NW
KTOP = 8
NEG = float("-inf")


def _tree_max(vs):
    vs = list(vs)
    while len(vs) > 1:
        nxt = [jnp.maximum(vs[i], vs[i + 1]) for i in range(0, len(vs) - 1, 2)]
        if len(vs) % 2:
            nxt.append(vs[-1])
        vs = nxt
    return vs[0]


def _sc_body(in_hbm, k_hbm, out_hbm, buf, obuf, kv_v, lsem):
    wid = lax.axis_index("s") * 2 + lax.axis_index("c")
    base = wid * RPW
    pltpu.sync_copy(k_hbm, kv_v)
    kv = kv_v[...]

    def row_fn(i, carry):
        row = base + i
        pltpu.async_copy(in_hbm.at[row], buf, lsem).wait()

        init = tuple(jnp.full((L,), NEG) for _ in range(2 * G))

        def p1(j, st):
            cs = list(st[:G])
            ds = list(st[G:])
            for g in range(G):
                v = buf[pl.ds(j * G * L + g * L, L)]
                lo = jnp.minimum(cs[g], v)
                cs[g] = jnp.maximum(cs[g], v)
                ds[g] = jnp.maximum(ds[g], lo)
            return tuple(cs) + tuple(ds)

        pool = lax.fori_loop(0, NV // G, p1, init)

        t = jnp.float32(float("inf"))
        for _ in range(KTOP):
            masked = [jnp.where(p < t, p, NEG) for p in pool]
            t = jnp.max(_tree_max(masked))
        t8v = jnp.full((L,), t)

        U = 8

        def p2(j, cnt):
            for u in range(U):
                off = (j * U + u) * L
                x = buf[pl.ds(off, L)]
                m = x >= t8v
                obuf[pl.ds(off, L)] = jnp.where(m, x, 0.0) + kv
                cnt = cnt + m.astype(jnp.int32)
            return cnt

        cnt = lax.fori_loop(0, NV // U, p2, jnp.zeros((L,), jnp.int32))
        count = jnp.sum(cnt)

        @pl.when(count != KTOP)
        def _fallback():
            def cond(st):
                return st[1] < KTOP

            def body(st):
                tc, cgt, _tp, _cp = st
                tcv = jnp.full((L,), tc)

                def pw(j, c2):
                    mv, ce = c2
                    for u in range(U):
                        x = buf[pl.ds((j * U + u) * L, L)]
                        mv = jnp.maximum(mv, jnp.where(x < tcv, x, NEG))
                        ce = ce + (x == tcv).astype(jnp.int32)
                    return (mv, ce)

                mv, ce = lax.fori_loop(
                    0, NV // U, pw,
                    (jnp.full((L,), NEG), jnp.zeros((L,), jnp.int32)))
                return (jnp.max(mv), cgt + jnp.sum(ce), tc, cgt)

            tinf = jnp.float32(float("inf"))
            st = lax.while_loop(
                cond, body, (tinf, jnp.int32(0), tinf, jnp.int32(0)))
            t8x = st[2]
            need = KTOP - st[3]
            t8xv = jnp.full((L,), t8x)

            def pr(j, before):
                for u in range(U):
                    off = (j * U + u) * L
                    x = buf[pl.ds(off, L)]
                    gt = x > t8xv
                    eq = x == t8xv
                    eqi = eq.astype(jnp.int32)
                    incl = lax.cumsum(eqi, axis=0)
                    keep = jnp.logical_or(
                        gt, jnp.logical_and(eq, (before + incl) <= need))
                    obuf[pl.ds(off, L)] = jnp.where(keep, x, 0.0) + kv
                    before = before + jnp.sum(eqi)
                return before

            lax.fori_loop(0, NV // U, pr, jnp.int32(0))

        pltpu.async_copy(obuf, out_hbm.at[row], lsem).wait()
        return carry

    lax.fori_loop(0, RPW, row_fn, jnp.int32(0))


def kernel(t, K):
    B, R, Cc = t.shape
    flat = t.reshape(B * R, Cc)
    kvec = jnp.full((L,), jnp.asarray(K, jnp.float32) - float(KTOP),
                    dtype=jnp.float32)
    mesh = plsc.VectorSubcoreMesh(core_axis_name="c", subcore_axis_name="s")
    out = pl.kernel(
        _sc_body,
        out_type=jax.ShapeDtypeStruct((B * R, Cc), jnp.float32),
        mesh=mesh,
        compiler_params=pltpu.CompilerParams(needs_layout_passes=False),
        scratch_types=[
            pltpu.VMEM((C,), jnp.float32),
            pltpu.VMEM((C,), jnp.float32),
            pltpu.VMEM((L,), jnp.float32),
            pltpu.SemaphoreType.DMA,
        ],
    )(flat, kvec)
    return out.reshape(B, R, Cc)


# speedup vs baseline: 10.1561x; 1.4539x over previous
import jax
import jax.numpy as jnp
from jax import lax
from jax.experimental import pallas as pl
from jax.experimental.pallas import tpu as pltpu
from jax.experimental.pallas import tpu_sc as plsc

L = 16
C = 32768
NV = C // L
G = 16
ROWS = 1024
NW = 32
RPW = ROWS // NW
KTOP = 8
NEG = float("-inf")


def _tree_max(vs):
    vs = list(vs)
    while len(vs) > 1:
        nxt = [jnp.maximum(vs[i], vs[i + 1]) for i in range(0, len(vs) - 1, 2)]
        if len(vs) % 2:
            nxt.append(vs[-1])
        vs = nxt
    return vs[0]


def _sc_body(in_hbm, k_hbm, out_hbm, bufA, bufB, kv_v, lsemA, lsemB, ssem):
    wid = lax.axis_index("s") * 2 + lax.axis_index("c")
    base = wid * RPW
    last = base + RPW - 1
    pltpu.sync_copy(k_hbm, kv_v)
    kv = kv_v[...]

    pltpu.async_copy(in_hbm.at[base], bufA, lsemA)
    pltpu.async_copy(in_hbm.at[base + 1], bufB, lsemB)

    def process(row, buf, lsem, other, olsem):
        pltpu.make_async_copy(in_hbm.at[row], buf, lsem).wait()

        init = tuple(jnp.full((L,), NEG) for _ in range(2 * G))

        def p1(j, st):
            cs = list(st[:G])
            ds = list(st[G:])
            for g in range(G):
                v = buf[pl.ds(j * G * L + g * L, L)]
                lo = jnp.minimum(cs[g], v)
                cs[g] = jnp.maximum(cs[g], v)
                ds[g] = jnp.maximum(ds[g], lo)
            return tuple(cs) + tuple(ds)

        pool = lax.fori_loop(0, NV // G, p1, init)

        t = jnp.float32(float("inf"))
        for _ in range(KTOP):
            masked = [jnp.where(p < t, p, NEG) for p in pool]
            t = jnp.max(_tree_max(masked))
        t8v = jnp.full((L,), t)

        @pl.when(row > base)
        def _pump():
            pltpu.make_async_copy(other, out_hbm.at[row], ssem).wait()
            nxt = jnp.minimum(row + 1, last)
            pltpu.async_copy(in_hbm.at[nxt], other, olsem)

        U = 8

        def p2(j, cnt):
            for u in range(U):
                off = (j * U + u) * L
                x = buf[pl.ds(off, L)]
                m = x >= t8v
                buf[pl.ds(off, L)] = jnp.where(m, x, 0.0) + kv
                cnt = cnt + m.astype(jnp.int32)
            return cnt

        cnt = lax.fori_loop(0, NV // U, p2, jnp.zeros((L,), jnp.int32))
        count = jnp.sum(cnt)

        @pl.when(count != KTOP)
        def _fallback():
            pltpu.sync_copy(in_hbm.at[row], buf)

            def cond(st):
                return st[1] < KTOP

            def body(st):
                tc, cgt, _tp, _cp = st
                tcv = jnp.full((L,), tc)

                def pw(j, c2):
                    mv, ce = c2
                    for u in range(U):
                        x = buf[pl.ds((j * U + u) * L, L)]
                        mv = jnp.maximum(mv, jnp.where(x < tcv, x, NEG))
                        ce = ce + (x == tcv).astype(jnp.int32)
                    return (mv, ce)

                mv, ce = lax.fori_loop(
                    0, NV // U, pw,
                    (jnp.full((L,), NEG), jnp.zeros((L,), jnp.int32)))
                return (jnp.max(mv), cgt + jnp.sum(ce), tc, cgt)

            tinf = jnp.float32(float("inf"))
            st = lax.while_loop(
                cond, body, (tinf, jnp.int32(0), tinf, jnp.int32(0)))
            t8x = st[2]
            need = KTOP - st[3]
            t8xv = jnp.full((L,), t8x)

            def pr(j, before):
                for u in range(U):
                    off = (j * U + u) * L
                    x = buf[pl.ds(off, L)]
                    gt = x > t8xv
                    eq = x == t8xv
                    eqi = eq.astype(jnp.int32)
                    incl = lax.cumsum(eqi, axis=0)
                    keep = jnp.logical_or(
                        gt, jnp.logical_and(eq, (before + incl) <= need))
                    buf[pl.ds(off, L)] = jnp.where(keep, x, 0.0) + kv
                    before = before + jnp.sum(eqi)
                return before

            lax.fori_loop(0, NV // U, pr, jnp.int32(0))

        pltpu.async_copy(buf, out_hbm.at[row], ssem)

    def pair(i, carry):
        process(base + 2 * i, bufA, lsemA, bufB, lsemB)
        process(base + 2 * i + 1, bufB, lsemB, bufA, lsemA)
        return carry

    lax.fori_loop(0, RPW // 2, pair, jnp.int32(0))

    pltpu.make_async_copy(bufB, out_hbm.at[last], ssem).wait()
    pltpu.make_async_copy(in_hbm.at[last], bufA, lsemA).wait()


def kernel(t, K):
    B, R, Cc = t.shape
    flat = t.reshape(B * R, Cc)
    kvec = jnp.full((L,), jnp.asarray(K, jnp.float32) - float(KTOP),
                    dtype=jnp.float32)
    mesh = plsc.VectorSubcoreMesh(core_axis_name="c", subcore_axis_name="s")
    out = pl.kernel(
        _sc_body,
        out_type=jax.ShapeDtypeStruct((B * R, Cc), jnp.float32),
        mesh=mesh,
        compiler_params=pltpu.CompilerParams(needs_layout_passes=False),
        scratch_types=[
            pltpu.VMEM((C,), jnp.float32),
            pltpu.VMEM((C,), jnp.float32),
            pltpu.VMEM((L,), jnp.float32),
            pltpu.SemaphoreType.DMA,
            pltpu.SemaphoreType.DMA,
            pltpu.SemaphoreType.DMA,
        ],
    )(flat, kvec)
    return out.reshape(B, R, Cc)
